# baseline (speedup 1.0000x reference)
import functools

import jax
import jax.numpy as jnp
from jax import lax
from jax.experimental import pallas as pl
from jax.experimental.pallas import tpu as pltpu
from jax.experimental.pallas import tpu_sc as plsc

BATCH = 16384
Z_DIM = 64
_NUM_CORES = 2
_NUM_SUBCORES = 16
_NW = _NUM_CORES * _NUM_SUBCORES
_BPW = BATCH // _NW
_L = 16


def _lookup_body(u_hbm, packed_hbm, out_mean, out_logvar,
                 idx_v, rows_v, sem):
  wid = lax.axis_index("s") * _NUM_CORES + lax.axis_index("c")
  base = wid * _BPW
  pltpu.sync_copy(u_hbm.at[pl.ds(base, _BPW)], idx_v)
  pltpu.async_copy(packed_hbm.at[idx_v], rows_v, sem).wait()

  cp_m = pltpu.async_copy(rows_v.at[:, pl.ds(0, Z_DIM)],
                          out_mean.at[pl.ds(base, _BPW)], sem)
  cp_l = pltpu.async_copy(rows_v.at[:, pl.ds(Z_DIM, Z_DIM)],
                          out_logvar.at[pl.ds(base, _BPW)], sem)
  cp_m.wait()
  cp_l.wait()


@jax.jit
def kernel(u, mean_table, logvar_table):
  mesh = plsc.VectorSubcoreMesh(core_axis_name="c", subcore_axis_name="s")
  out = jax.ShapeDtypeStruct((BATCH, Z_DIM), jnp.float32)
  packed = jnp.concatenate([mean_table, logvar_table], axis=1)
  run = pl.kernel(
      _lookup_body,
      out_type=(out, out),
      mesh=mesh,
      scratch_types=[
          pltpu.VMEM((_BPW,), jnp.int32),
          pltpu.VMEM((_BPW, 2 * Z_DIM), jnp.float32),
          pltpu.SemaphoreType.DMA,
      ],
      compiler_params=pltpu.CompilerParams(
          use_tc_tiling_on_sc=False,
          skip_device_barrier=True,
          disable_bounds_checks=True,
          disable_semaphore_checks=True,
      ),
  )
  return run(u.astype(jnp.int32), packed)


# speedup vs baseline: 1.0099x; 1.0099x over previous
import functools

import jax
import jax.numpy as jnp
from jax import lax
from jax.experimental import pallas as pl
from jax.experimental.pallas import tpu as pltpu
from jax.experimental.pallas import tpu_sc as plsc

BATCH = 16384
Z_DIM = 64
_NUM_CORES = 2
_NUM_SUBCORES = 16
_NW = _NUM_CORES * _NUM_SUBCORES
_BPW = BATCH // _NW
_L = 16


_H = _BPW // 2


def _lookup_body(u_hbm, packed_hbm, out_mean, out_logvar,
                 idx_v, rows_v, gsem, wsem):
  wid = lax.axis_index("s") * _NUM_CORES + lax.axis_index("c")
  base = wid * _BPW
  pltpu.sync_copy(u_hbm.at[pl.ds(base, _BPW)], idx_v)

  cps = [pltpu.async_copy(packed_hbm.at[idx_v.at[pl.ds(h * _H, _H)]],
                          rows_v.at[pl.ds(h * _H, _H)], gsem)
         for h in range(2)]
  for h in range(2):
    cps[h].wait()
    sl = pl.ds(h * _H, _H)
    out_sl = pl.ds(base + h * _H, _H)
    pltpu.async_copy(rows_v.at[sl, pl.ds(0, Z_DIM)], out_mean.at[out_sl], wsem)
    pltpu.async_copy(rows_v.at[sl, pl.ds(Z_DIM, Z_DIM)],
                     out_logvar.at[out_sl], wsem)
  for h in range(2):
    sl = pl.ds(h * _H, _H)
    out_sl = pl.ds(base + h * _H, _H)
    pltpu.make_async_copy(rows_v.at[sl, pl.ds(0, Z_DIM)],
                          out_mean.at[out_sl], wsem).wait()
    pltpu.make_async_copy(rows_v.at[sl, pl.ds(Z_DIM, Z_DIM)],
                          out_logvar.at[out_sl], wsem).wait()


@jax.jit
def kernel(u, mean_table, logvar_table):
  mesh = plsc.VectorSubcoreMesh(core_axis_name="c", subcore_axis_name="s")
  out = jax.ShapeDtypeStruct((BATCH, Z_DIM), jnp.float32)
  packed = jnp.concatenate([mean_table, logvar_table], axis=1)
  run = pl.kernel(
      _lookup_body,
      out_type=(out, out),
      mesh=mesh,
      scratch_types=[
          pltpu.VMEM((_BPW,), jnp.int32),
          pltpu.VMEM((_BPW, 2 * Z_DIM), jnp.float32),
          pltpu.SemaphoreType.DMA,
          pltpu.SemaphoreType.DMA,
      ],
      compiler_params=pltpu.CompilerParams(
          use_tc_tiling_on_sc=False,
          skip_device_barrier=True,
          disable_bounds_checks=True,
          disable_semaphore_checks=True,
      ),
  )
  return run(u.astype(jnp.int32), packed)


# speedup vs baseline: 1.0111x; 1.0013x over previous
import jax
import jax.numpy as jnp
from jax import lax
from jax.experimental import pallas as pl
from jax.experimental.pallas import tpu as pltpu
from jax.experimental.pallas import tpu_sc as plsc

BATCH = 16384
Z_DIM = 64
_NUM_CORES = 2
_NUM_SUBCORES = 16
_NW = _NUM_CORES * _NUM_SUBCORES
_BPW = BATCH // _NW
_H = _BPW // 2


def _lookup_body(u_hbm, packed_hbm, out_mean, out_logvar,
                 idx_v, rows_v, gsem, wsem):
  wid = lax.axis_index("s") * _NUM_CORES + lax.axis_index("c")
  base = wid * _BPW
  pltpu.sync_copy(u_hbm.at[pl.ds(base, _BPW)], idx_v)

  cps = [pltpu.async_copy(packed_hbm.at[idx_v.at[pl.ds(h * _H, _H)]],
                          rows_v.at[pl.ds(h * _H, _H)], gsem)
         for h in range(2)]
  for h in range(2):
    cps[h].wait()
    sl = pl.ds(h * _H, _H)
    out_sl = pl.ds(base + h * _H, _H)
    pltpu.async_copy(rows_v.at[sl, pl.ds(0, Z_DIM)], out_mean.at[out_sl], wsem)
    pltpu.async_copy(rows_v.at[sl, pl.ds(Z_DIM, Z_DIM)],
                     out_logvar.at[out_sl], wsem)
  for h in range(2):
    sl = pl.ds(h * _H, _H)
    out_sl = pl.ds(base + h * _H, _H)
    pltpu.make_async_copy(rows_v.at[sl, pl.ds(0, Z_DIM)],
                          out_mean.at[out_sl], wsem).wait()
    pltpu.make_async_copy(rows_v.at[sl, pl.ds(Z_DIM, Z_DIM)],
                          out_logvar.at[out_sl], wsem).wait()


@jax.jit
def kernel(u, mean_table, logvar_table):
  mesh = plsc.VectorSubcoreMesh(core_axis_name="c", subcore_axis_name="s")
  out = jax.ShapeDtypeStruct((BATCH, Z_DIM), jnp.float32)
  packed = jnp.concatenate([mean_table, logvar_table], axis=1)
  run = pl.kernel(
      _lookup_body,
      out_type=(out, out),
      mesh=mesh,
      scratch_types=[
          pltpu.VMEM((_BPW,), jnp.int32),
          pltpu.VMEM((_BPW, 2 * Z_DIM), jnp.float32),
          pltpu.SemaphoreType.DMA,
          pltpu.SemaphoreType.DMA,
      ],
      compiler_params=pltpu.CompilerParams(use_tc_tiling_on_sc=False),
  )
  return run(u.astype(jnp.int32), packed)
